# Initial kernel scaffold; baseline (speedup 1.0000x reference)
import jax
import jax.numpy as jnp
from jax.experimental import pallas as pl


def kernel(pred_labels, gt_labels):
    raise NotImplementedError("write your pallas kernel here")


# speedup vs baseline: 1.9325x; 1.9325x over previous
import jax
import jax.numpy as jnp
from jax import lax
from jax.experimental import pallas as pl
from jax.experimental.pallas import tpu as pltpu
from jax.experimental.pallas import tpu_sc as plsc

_C = 150
_NBINS = _C * _C
_BINS_PAD = 22528
_N = 2097152
_NC = 2
_NS = 16
_L = 16
_NW = _NC * _NS
_PER_W = _N // _NW
_CHUNK = 8192
_NCHUNK = _PER_W // _CHUNK
_SLICE = _BINS_PAD // _NS
_U = 8


def _hist_body(pred_hbm, gt_hbm, out_hbm, pred_v, gt_v, hist_v, red_v,
               acc_v, shared, sem):
    c = lax.axis_index("c")
    s = lax.axis_index("s")
    wid = c * _NS + s
    base = wid * _PER_W

    zero = jnp.zeros((_L,), jnp.int32)

    def zero_body(i, carry):
        hist_v[pl.ds(i * _L, _L)] = zero
        return carry

    lax.fori_loop(0, _BINS_PAD // _L, zero_body, 0)

    def issue(k, buf):
        src = pl.ds(base + k * _CHUNK, _CHUNK)
        return (pltpu.async_copy(pred_hbm.at[src], pred_v.at[buf], sem),
                pltpu.async_copy(gt_hbm.at[src], gt_v.at[buf], sem))

    pending = issue(0, 0)
    for k in range(_NCHUNK):
        buf = k % 2
        pending[0].wait()
        pending[1].wait()
        if k + 1 < _NCHUNK:
            pending = issue(k + 1, (k + 1) % 2)

        def acc_body(i, carry):
            off = i * (_L * _U)
            ones = jnp.ones((_L,), jnp.int32)
            ps = [pred_v[buf, pl.ds(off + u * _L, _L)] for u in range(_U)]
            gs = [gt_v[buf, pl.ds(off + u * _L, _L)] for u in range(_U)]
            idxs = [g * _C + p for p, g in zip(ps, gs)]
            for idx in idxs:
                plsc.addupdate_scatter(hist_v, [idx], ones)
            return carry

        lax.fori_loop(0, _CHUNK // (_L * _U), acc_body, 0)

    pltpu.sync_copy(hist_v, shared.at[s])
    plsc.subcore_barrier()
    for t in range(_NS):
        pltpu.sync_copy(shared.at[t, pl.ds(s * _SLICE, _SLICE)], red_v.at[t])

    def red_body(j, carry):
        v = red_v[0, pl.ds(j * _L, _L)]
        for t in range(1, _NS):
            v = v + red_v[t, pl.ds(j * _L, _L)]
        acc_v[pl.ds(j * _L, _L)] = v
        return carry

    lax.fori_loop(0, _SLICE // _L, red_body, 0)
    pltpu.sync_copy(acc_v, out_hbm.at[c, pl.ds(s * _SLICE, _SLICE)])


_sc_hist = pl.kernel(
    _hist_body,
    out_type=jax.ShapeDtypeStruct((_NC, _BINS_PAD), jnp.int32),
    mesh=plsc.VectorSubcoreMesh(core_axis_name="c", subcore_axis_name="s"),
    compiler_params=pltpu.CompilerParams(needs_layout_passes=False),
    scratch_types=[
        pltpu.VMEM((2, _CHUNK), jnp.int32),
        pltpu.VMEM((2, _CHUNK), jnp.int32),
        pltpu.VMEM((_BINS_PAD,), jnp.int32),
        pltpu.VMEM((_NS, _SLICE), jnp.int32),
        pltpu.VMEM((_SLICE,), jnp.int32),
        pltpu.VMEM_SHARED((_NS, _BINS_PAD), jnp.int32),
        pltpu.SemaphoreType.DMA,
    ],
)


def _fin_body(m_ref, conf_ref, iou_ref, acc_ref):
    m = m_ref[...].astype(jnp.float32)
    conf = m[0] + m[1]
    rows = jnp.sum(conf, axis=1)
    cols = jnp.sum(conf, axis=0)
    ii = lax.broadcasted_iota(jnp.int32, (_C, _C), 0)
    jj = lax.broadcasted_iota(jnp.int32, (_C, _C), 1)
    tp = jnp.sum(jnp.where(ii == jj, conf, 0.0), axis=1)
    fp = rows - tp
    fn = cols - tp
    conf_ref[...] = conf
    iou_ref[...] = tp / (tp + fp + fn + 1e-15)
    acc_ref[...] = tp / (tp + fp)


_fin = pl.pallas_call(
    _fin_body,
    out_shape=(
        jax.ShapeDtypeStruct((_C, _C), jnp.float32),
        jax.ShapeDtypeStruct((_C,), jnp.float32),
        jax.ShapeDtypeStruct((_C,), jnp.float32),
    ),
)


@jax.jit
def kernel(pred_labels, gt_labels):
    parts = _sc_hist(pred_labels.astype(jnp.int32), gt_labels.astype(jnp.int32))
    m = parts[:, :_NBINS].reshape(_NC, _C, _C)
    return _fin(m)
